# baseline (speedup 1.0000x reference)
import functools

import jax
import jax.numpy as jnp
from jax import lax
from jax.experimental import pallas as pl
from jax.experimental.pallas import tpu as pltpu
from jax.experimental.pallas import tpu_sc as plsc

NPIX = 196608
NFREQS = 128
NPTS = 65536

NW = 32
PTS_PER_W = NPTS // NW
PTS_PER_SUB = 32
ROWS_PER_SUB = PTS_PER_SUB * 4
SUBS = PTS_PER_W // PTS_PER_SUB
NBUF = 4
LANES = 16
SLICES = NFREQS // LANES


def _sc_gather(table, idx3, wgt3):
    mesh = plsc.VectorSubcoreMesh(core_axis_name="c", subcore_axis_name="s")

    @functools.partial(
        pl.kernel,
        out_type=jax.ShapeDtypeStruct((NPTS, NFREQS), jnp.float32),
        mesh=mesh,
        scratch_types=[
            pltpu.VMEM((SUBS, ROWS_PER_SUB), jnp.int32),
            pltpu.VMEM((SUBS, ROWS_PER_SUB), jnp.float32),
            pltpu.VMEM((NBUF, ROWS_PER_SUB, NFREQS), jnp.float32),
            pltpu.VMEM((2, PTS_PER_SUB, NFREQS), jnp.float32),
            pltpu.SemaphoreType.DMA,
            pltpu.SemaphoreType.DMA,
            pltpu.SemaphoreType.DMA,
            pltpu.SemaphoreType.DMA,
            pltpu.SemaphoreType.DMA,
            pltpu.SemaphoreType.DMA,
        ],
    )
    def k(table_hbm, idx_hbm, wgt_hbm, out_hbm, idx_v, wgt_v, buf, outb,
          gsem0, gsem1, gsem2, gsem3, osem0, osem1):
        gsems = (gsem0, gsem1, gsem2, gsem3)
        osems = (osem0, osem1)
        wid = lax.axis_index("s") * 2 + lax.axis_index("c")
        base = wid * PTS_PER_W
        pltpu.sync_copy(idx_hbm.at[wid], idx_v)
        pltpu.sync_copy(wgt_hbm.at[wid], wgt_v)

        for u in range(NBUF):
            pltpu.async_copy(table_hbm.at[idx_v.at[u]], buf.at[u], gsems[u])

        def quad_group(tq, carry):
            for u in range(NBUF):
                g = NBUF * tq + u
                ou = u % 2
                pltpu.make_async_copy(
                    table_hbm.at[idx_v.at[g]], buf.at[u], gsems[u]
                ).wait()

                def _wait_out():
                    pltpu.make_async_copy(
                        outb.at[ou],
                        out_hbm.at[pl.ds(base + (g - 2) * PTS_PER_SUB,
                                         PTS_PER_SUB)],
                        osems[ou],
                    ).wait()

                if u < 2:
                    pl.when(tq >= 1)(_wait_out)
                else:
                    _wait_out()

                def quad_body(q, c, u=u, ou=ou):
                    wv = wgt_v[g, pl.ds(q * LANES, LANES)]
                    for pp in range(4):
                        p = q * 4 + pp
                        w = [
                            jnp.full((LANES,), wv[4 * pp + j],
                                     dtype=jnp.float32)
                            for j in range(4)
                        ]
                        for s in range(SLICES):
                            acc = w[0] * buf[u, 4 * p + 0,
                                             pl.ds(s * LANES, LANES)]
                            for j in range(1, 4):
                                acc = acc + w[j] * buf[u, 4 * p + j,
                                                       pl.ds(s * LANES, LANES)]
                            outb[ou, p, pl.ds(s * LANES, LANES)] = acc
                    return c

                lax.fori_loop(0, PTS_PER_SUB // 4, quad_body, 0, unroll=False)

                @pl.when(g + NBUF < SUBS)
                def _():
                    pltpu.async_copy(
                        table_hbm.at[idx_v.at[g + NBUF]], buf.at[u], gsems[u]
                    )

                pltpu.async_copy(
                    outb.at[ou],
                    out_hbm.at[pl.ds(base + g * PTS_PER_SUB, PTS_PER_SUB)],
                    osems[ou],
                )
            return carry

        lax.fori_loop(0, SUBS // NBUF, quad_group, 0, unroll=False)

        for ou in range(2):
            pltpu.make_async_copy(
                outb.at[ou],
                out_hbm.at[pl.ds(base + (SUBS - 2 + ou) * PTS_PER_SUB,
                                 PTS_PER_SUB)],
                osems[ou],
            ).wait()

    return k(table, idx3, wgt3)


def kernel(params, inds, wgts, freqs):
    table = params.reshape(NFREQS, NPIX).T
    idx3 = inds.astype(jnp.int32).reshape(NW, SUBS, ROWS_PER_SUB)
    wgt3 = wgts.astype(jnp.float32).reshape(NW, SUBS, ROWS_PER_SUB)
    out = _sc_gather(table, idx3, wgt3)
    return out.T.reshape(1, 1, 1, NFREQS, NPTS)
